# baseline (speedup 1.0000x reference)
import functools

import jax
import jax.numpy as jnp
from jax import lax
from jax.experimental import pallas as pl
from jax.experimental.pallas import tpu as pltpu
from jax.experimental.pallas import tpu_sc as plsc

D = 100
DP = 128
NC = 2
NS = 16
NW = NC * NS
G = 128
K = 2
CHUNK = K * G
VB = 512


def _eye(n):
    return (jax.lax.broadcasted_iota(jnp.int32, (n, n), 0)
            == jax.lax.broadcasted_iota(jnp.int32, (n, n), 1)
            ).astype(jnp.float32)


def _table_prep(wt, vp):
    def body(wt_ref, out_ref):
        ey = _eye(DP)
        blk = jnp.concatenate(
            [wt_ref[...], jnp.zeros((DP - D, VB), jnp.float32)], axis=0)
        out_ref[...] = jax.lax.dot_general(
            blk, ey, (((0,), (0,)), ((), ())),
            preferred_element_type=jnp.float32,
            precision=jax.lax.Precision.HIGHEST)

    return pl.pallas_call(
        body,
        grid=(vp // VB,),
        in_specs=[pl.BlockSpec((D, VB), lambda i: (0, i))],
        out_specs=pl.BlockSpec((VB, DP), lambda i: (i, 0)),
        out_shape=jax.ShapeDtypeStruct((vp, DP), jnp.float32),
    )(wt)


def _out_prep(x, hist, batch):
    def body(x_ref, o_ref):
        ey = _eye(DP)
        c = jax.lax.dot_general(
            ey, x_ref[0], (((1,), (1,)), ((), ())),
            preferred_element_type=jnp.float32,
            precision=jax.lax.Precision.HIGHEST)
        o_ref[...] = c.reshape(DP, batch // 128, 128)[:D].reshape(
            D * batch // 128, 128)

    rows = D * batch // 128
    return pl.pallas_call(
        body,
        grid=(hist,),
        in_specs=[pl.BlockSpec((1, batch, DP), lambda l: (l, 0, 0))],
        out_specs=pl.BlockSpec((rows, 128), lambda l: (l, 0)),
        out_shape=jax.ShapeDtypeStruct((hist * rows, 128), jnp.float32),
    )(x)


def _make_gather(n_rows):
    rows_per_w = n_rows // NW
    ng = rows_per_w // G
    nchunk = ng // K
    assert n_rows % (NW * G) == 0 and ng % (2 * K) == 0

    mesh = plsc.VectorSubcoreMesh(core_axis_name="c", subcore_axis_name="s")

    @functools.partial(
        pl.kernel,
        out_type=jax.ShapeDtypeStruct((n_rows, DP), jnp.float32),
        mesh=mesh,
        scratch_types=[
            pltpu.VMEM((ng, G), jnp.int32),
            pltpu.VMEM((CHUNK, DP), jnp.float32),
            pltpu.VMEM((CHUNK, DP), jnp.float32),
            pltpu.SemaphoreType.DMA,
            pltpu.SemaphoreType.DMA,
            pltpu.SemaphoreType.DMA,
            pltpu.SemaphoreType.DMA,
        ],
    )
    def gather_kernel(idx_hbm, table_hbm, out_hbm,
                      idx_v, buf0, buf1, gsem0, gsem1, wsem0, wsem1):
        wid = lax.axis_index("s") * NC + lax.axis_index("c")
        row0 = wid * rows_per_w

        pltpu.sync_copy(idx_hbm.at[wid], idx_v)

        def out_slice(c):
            return out_hbm.at[pl.ds(row0 + c * CHUNK, CHUNK)]

        def fire_gathers(c, buf, sem):
            return [
                pltpu.async_copy(
                    table_hbm.at[idx_v.at[c * K + j]],
                    buf.at[pl.ds(j * G, G)],
                    sem,
                )
                for j in range(K)
            ]

        def body(t, _):
            a = 2 * t

            @pl.when(t > 0)
            def _drain_prev():
                pltpu.make_async_copy(buf0, out_slice(a - 2), wsem0).wait()
                pltpu.make_async_copy(buf1, out_slice(a - 1), wsem1).wait()

            ha = fire_gathers(a, buf0, gsem0)
            hb = fire_gathers(a + 1, buf1, gsem1)
            for h in ha:
                h.wait()
            pltpu.async_copy(buf0, out_slice(a), wsem0)
            for h in hb:
                h.wait()
            pltpu.async_copy(buf1, out_slice(a + 1), wsem1)
            return 0

        lax.fori_loop(0, nchunk // 2, body, 0)
        pltpu.make_async_copy(buf0, out_slice(nchunk - 2), wsem0).wait()
        pltpu.make_async_copy(buf1, out_slice(nchunk - 1), wsem1).wait()

    return gather_kernel


def kernel(sen, word_embeddings):
    batch, hist = sen.shape
    vocab = word_embeddings.shape[0]
    vp = -(-vocab // VB) * VB
    n_rows = batch * hist
    rows_per_w = n_rows // NW

    idx = jnp.transpose(sen).reshape(NW, rows_per_w // G, G)
    table = _table_prep(jnp.transpose(word_embeddings), vp)
    out = _make_gather(n_rows)(idx, table)
    flat = _out_prep(out.reshape(hist, batch, DP), hist, batch)
    y = flat.reshape(hist, D, batch // 128, 128)
    y = jnp.transpose(y, (2, 3, 0, 1))
    return y.reshape(batch, hist, D, 1)


# speedup vs baseline: 1.1947x; 1.1947x over previous
import functools

import jax
import jax.numpy as jnp
from jax import lax
from jax.experimental import pallas as pl
from jax.experimental.pallas import tpu as pltpu
from jax.experimental.pallas import tpu_sc as plsc

D = 100
DP = 128
NC = 2
NS = 16
NW = NC * NS
G = 128
K = 2
CHUNK = K * G
VB = 512


def _table_prep(wt, vp):
    def body(wt_ref, out_ref):
        blk = jnp.concatenate(
            [wt_ref[...], jnp.zeros((DP - D, VB), jnp.float32)], axis=0)
        out_ref[...] = jnp.transpose(blk)

    return pl.pallas_call(
        body,
        grid=(vp // VB,),
        in_specs=[pl.BlockSpec((D, VB), lambda i: (0, i))],
        out_specs=pl.BlockSpec((VB, DP), lambda i: (i, 0)),
        out_shape=jax.ShapeDtypeStruct((vp, DP), jnp.float32),
    )(wt)


def _out_prep(x, hist, batch):
    def body(x_ref, o_ref):
        c = jnp.transpose(x_ref[0])
        o_ref[...] = c.reshape(DP, batch // 128, 128)[:D].reshape(
            D * batch // 128, 128)

    rows = D * batch // 128
    return pl.pallas_call(
        body,
        grid=(hist,),
        in_specs=[pl.BlockSpec((1, batch, DP), lambda l: (l, 0, 0))],
        out_specs=pl.BlockSpec((rows, 128), lambda l: (l, 0)),
        out_shape=jax.ShapeDtypeStruct((hist * rows, 128), jnp.float32),
    )(x)


def _make_gather(n_rows):
    rows_per_w = n_rows // NW
    ng = rows_per_w // G
    nchunk = ng // K
    assert n_rows % (NW * G) == 0 and ng % (2 * K) == 0

    mesh = plsc.VectorSubcoreMesh(core_axis_name="c", subcore_axis_name="s")

    @functools.partial(
        pl.kernel,
        out_type=jax.ShapeDtypeStruct((n_rows, DP), jnp.float32),
        mesh=mesh,
        scratch_types=[
            pltpu.VMEM((ng, G), jnp.int32),
            pltpu.VMEM((CHUNK, DP), jnp.float32),
            pltpu.VMEM((CHUNK, DP), jnp.float32),
            pltpu.SemaphoreType.DMA,
            pltpu.SemaphoreType.DMA,
            pltpu.SemaphoreType.DMA,
            pltpu.SemaphoreType.DMA,
        ],
    )
    def gather_kernel(idx_hbm, table_hbm, out_hbm,
                      idx_v, buf0, buf1, gsem0, gsem1, wsem0, wsem1):
        wid = lax.axis_index("s") * NC + lax.axis_index("c")
        row0 = wid * rows_per_w

        pltpu.sync_copy(idx_hbm.at[wid], idx_v)

        def out_slice(c):
            return out_hbm.at[pl.ds(row0 + c * CHUNK, CHUNK)]

        def fire_gathers(c, buf, sem):
            return [
                pltpu.async_copy(
                    table_hbm.at[idx_v.at[c * K + j]],
                    buf.at[pl.ds(j * G, G)],
                    sem,
                )
                for j in range(K)
            ]

        def body(t, _):
            a = 2 * t

            @pl.when(t > 0)
            def _drain_prev():
                pltpu.make_async_copy(buf0, out_slice(a - 2), wsem0).wait()
                pltpu.make_async_copy(buf1, out_slice(a - 1), wsem1).wait()

            ha = fire_gathers(a, buf0, gsem0)
            hb = fire_gathers(a + 1, buf1, gsem1)
            for h in ha:
                h.wait()
            pltpu.async_copy(buf0, out_slice(a), wsem0)
            for h in hb:
                h.wait()
            pltpu.async_copy(buf1, out_slice(a + 1), wsem1)
            return 0

        lax.fori_loop(0, nchunk // 2, body, 0)
        pltpu.make_async_copy(buf0, out_slice(nchunk - 2), wsem0).wait()
        pltpu.make_async_copy(buf1, out_slice(nchunk - 1), wsem1).wait()

    return gather_kernel


def kernel(sen, word_embeddings):
    batch, hist = sen.shape
    vocab = word_embeddings.shape[0]
    vp = -(-vocab // VB) * VB
    n_rows = batch * hist
    rows_per_w = n_rows // NW

    idx = jnp.transpose(sen).reshape(NW, rows_per_w // G, G)
    table = _table_prep(jnp.transpose(word_embeddings), vp)
    out = _make_gather(n_rows)(idx, table)
    flat = _out_prep(out.reshape(hist, batch, DP), hist, batch)
    y = flat.reshape(hist, D, batch // 128, 128)
    y = jnp.transpose(y, (2, 3, 0, 1))
    return y.reshape(batch, hist, D, 1)


# speedup vs baseline: 1.5198x; 1.2721x over previous
import functools

import jax
import jax.numpy as jnp
from jax import lax
from jax.experimental import pallas as pl
from jax.experimental.pallas import tpu as pltpu
from jax.experimental.pallas import tpu_sc as plsc

D = 100
DP = 128
NC = 2
NS = 16
NW = NC * NS
G = 128
K = 2
CHUNK = K * G
VB = 2048


def _table_prep(wt, vp):
    def body(wt_ref, out_ref):
        blk = jnp.concatenate(
            [wt_ref[...], jnp.zeros((DP - D, VB), jnp.float32)], axis=0)
        out_ref[...] = jnp.transpose(blk)

    return pl.pallas_call(
        body,
        grid=(vp // VB,),
        in_specs=[pl.BlockSpec((D, VB), lambda i: (0, i))],
        out_specs=pl.BlockSpec((VB, DP), lambda i: (i, 0)),
        out_shape=jax.ShapeDtypeStruct((vp, DP), jnp.float32),
    )(wt)


def _out_prep(x, hist, batch):
    def body(x_ref, o_ref):
        c = jnp.transpose(x_ref[0])
        o_ref[...] = c.reshape(DP, batch // 128, 128)[:D].reshape(
            D * batch // 128, 128)

    rows = D * batch // 128
    return pl.pallas_call(
        body,
        grid=(hist,),
        in_specs=[pl.BlockSpec((1, batch, DP), lambda l: (l, 0, 0))],
        out_specs=pl.BlockSpec((rows, 128), lambda l: (l, 0)),
        out_shape=jax.ShapeDtypeStruct((hist * rows, 128), jnp.float32),
    )(x)


def _make_gather(n_rows):
    rows_per_w = n_rows // NW
    ng = rows_per_w // G
    nchunk = ng // K
    assert n_rows % (NW * G) == 0 and ng % (2 * K) == 0

    mesh = plsc.VectorSubcoreMesh(core_axis_name="c", subcore_axis_name="s")

    @functools.partial(
        pl.kernel,
        out_type=jax.ShapeDtypeStruct((n_rows, DP), jnp.float32),
        mesh=mesh,
        scratch_types=[
            pltpu.VMEM((ng, G), jnp.int32),
            pltpu.VMEM((CHUNK, DP), jnp.float32),
            pltpu.VMEM((CHUNK, DP), jnp.float32),
            pltpu.SemaphoreType.DMA,
            pltpu.SemaphoreType.DMA,
            pltpu.SemaphoreType.DMA,
            pltpu.SemaphoreType.DMA,
        ],
    )
    def gather_kernel(idx_hbm, table_hbm, out_hbm,
                      idx_v, buf0, buf1, gsem0, gsem1, wsem0, wsem1):
        wid = lax.axis_index("s") * NC + lax.axis_index("c")
        row0 = wid * rows_per_w

        pltpu.sync_copy(idx_hbm.at[wid], idx_v)

        def out_slice(c):
            return out_hbm.at[pl.ds(row0 + c * CHUNK, CHUNK)]

        def fire_gathers(c, buf, sem):
            return [
                pltpu.async_copy(
                    table_hbm.at[idx_v.at[c * K + j]],
                    buf.at[pl.ds(j * G, G)],
                    sem,
                )
                for j in range(K)
            ]

        def body(t, _):
            a = 2 * t

            @pl.when(t > 0)
            def _drain_prev():
                pltpu.make_async_copy(buf0, out_slice(a - 2), wsem0).wait()
                pltpu.make_async_copy(buf1, out_slice(a - 1), wsem1).wait()

            ha = fire_gathers(a, buf0, gsem0)
            hb = fire_gathers(a + 1, buf1, gsem1)
            for h in ha:
                h.wait()
            pltpu.async_copy(buf0, out_slice(a), wsem0)
            for h in hb:
                h.wait()
            pltpu.async_copy(buf1, out_slice(a + 1), wsem1)
            return 0

        lax.fori_loop(0, nchunk // 2, body, 0)
        pltpu.make_async_copy(buf0, out_slice(nchunk - 2), wsem0).wait()
        pltpu.make_async_copy(buf1, out_slice(nchunk - 1), wsem1).wait()

    return gather_kernel


def kernel(sen, word_embeddings):
    batch, hist = sen.shape
    vocab = word_embeddings.shape[0]
    vp = -(-vocab // VB) * VB
    n_rows = batch * hist
    rows_per_w = n_rows // NW

    idx = jnp.transpose(sen).reshape(NW, rows_per_w // G, G)
    table = _table_prep(jnp.transpose(word_embeddings), vp)
    out = _make_gather(n_rows)(idx, table)
    flat = _out_prep(out.reshape(hist, batch, DP), hist, batch)
    y = flat.reshape(hist, D, batch // 128, 128)
    y = jnp.transpose(y, (2, 3, 0, 1))
    return y.reshape(batch, hist, D, 1)


# speedup vs baseline: 1.7382x; 1.1437x over previous
import functools

import jax
import jax.numpy as jnp
from jax import lax
from jax.experimental import pallas as pl
from jax.experimental.pallas import tpu as pltpu
from jax.experimental.pallas import tpu_sc as plsc

D = 100
DP = 128
NC = 2
NS = 16
NW = NC * NS
G = 128
K = 2
CHUNK = K * G
VB = 4096


def _table_prep(wt, vp):
    def body(wt_ref, out_ref):
        blk = jnp.concatenate(
            [wt_ref[...], jnp.zeros((DP - D, VB), jnp.float32)], axis=0)
        out_ref[...] = jnp.transpose(blk)

    return pl.pallas_call(
        body,
        grid=(vp // VB,),
        in_specs=[pl.BlockSpec((D, VB), lambda i: (0, i))],
        out_specs=pl.BlockSpec((VB, DP), lambda i: (i, 0)),
        out_shape=jax.ShapeDtypeStruct((vp, DP), jnp.float32),
    )(wt)


def _out_prep(x, hist, batch):
    rows = D * batch // 128
    lb = 2

    def body(x_ref, o_ref):
        for i in range(lb):
            c = jnp.transpose(x_ref[i])
            o_ref[pl.ds(i * rows, rows), :] = c.reshape(
                DP, batch // 128, 128)[:D].reshape(rows, 128)

    return pl.pallas_call(
        body,
        grid=(hist // lb,),
        in_specs=[pl.BlockSpec((lb, batch, DP), lambda l: (l, 0, 0))],
        out_specs=pl.BlockSpec((lb * rows, 128), lambda l: (l, 0)),
        out_shape=jax.ShapeDtypeStruct((hist * rows, 128), jnp.float32),
    )(x)


def _make_gather(n_rows):
    rows_per_w = n_rows // NW
    ng = rows_per_w // G
    nchunk = ng // K
    assert n_rows % (NW * G) == 0 and ng % (2 * K) == 0

    mesh = plsc.VectorSubcoreMesh(core_axis_name="c", subcore_axis_name="s")

    @functools.partial(
        pl.kernel,
        out_type=jax.ShapeDtypeStruct((n_rows, DP), jnp.float32),
        mesh=mesh,
        scratch_types=[
            pltpu.VMEM((ng, G), jnp.int32),
            pltpu.VMEM((CHUNK, DP), jnp.float32),
            pltpu.VMEM((CHUNK, DP), jnp.float32),
            pltpu.SemaphoreType.DMA,
            pltpu.SemaphoreType.DMA,
            pltpu.SemaphoreType.DMA,
            pltpu.SemaphoreType.DMA,
        ],
    )
    def gather_kernel(idx_hbm, table_hbm, out_hbm,
                      idx_v, buf0, buf1, gsem0, gsem1, wsem0, wsem1):
        wid = lax.axis_index("s") * NC + lax.axis_index("c")
        row0 = wid * rows_per_w

        pltpu.sync_copy(idx_hbm.at[wid], idx_v)

        def out_slice(c):
            return out_hbm.at[pl.ds(row0 + c * CHUNK, CHUNK)]

        def fire_gathers(c, buf, sem):
            return [
                pltpu.async_copy(
                    table_hbm.at[idx_v.at[c * K + j]],
                    buf.at[pl.ds(j * G, G)],
                    sem,
                )
                for j in range(K)
            ]

        def body(t, _):
            a = 2 * t

            @pl.when(t > 0)
            def _drain_prev():
                pltpu.make_async_copy(buf0, out_slice(a - 2), wsem0).wait()
                pltpu.make_async_copy(buf1, out_slice(a - 1), wsem1).wait()

            ha = fire_gathers(a, buf0, gsem0)
            hb = fire_gathers(a + 1, buf1, gsem1)
            for h in ha:
                h.wait()
            pltpu.async_copy(buf0, out_slice(a), wsem0)
            for h in hb:
                h.wait()
            pltpu.async_copy(buf1, out_slice(a + 1), wsem1)
            return 0

        lax.fori_loop(0, nchunk // 2, body, 0)
        pltpu.make_async_copy(buf0, out_slice(nchunk - 2), wsem0).wait()
        pltpu.make_async_copy(buf1, out_slice(nchunk - 1), wsem1).wait()

    return gather_kernel


def kernel(sen, word_embeddings):
    batch, hist = sen.shape
    vocab = word_embeddings.shape[0]
    vp = -(-vocab // VB) * VB
    n_rows = batch * hist
    rows_per_w = n_rows // NW

    idx = jnp.transpose(sen).reshape(NW, rows_per_w // G, G)
    table = _table_prep(jnp.transpose(word_embeddings), vp)
    out = _make_gather(n_rows)(idx, table)
    flat = _out_prep(out.reshape(hist, batch, DP), hist, batch)
    y = flat.reshape(hist, D, batch // 128, 128)
    y = jnp.transpose(y, (2, 3, 0, 1))
    return y.reshape(batch, hist, D, 1)


# speedup vs baseline: 1.8322x; 1.0541x over previous
import functools

import jax
import jax.numpy as jnp
from jax import lax
from jax.experimental import pallas as pl
from jax.experimental.pallas import tpu as pltpu
from jax.experimental.pallas import tpu_sc as plsc

D = 100
DP = 128
NC = 2
NS = 16
NW = NC * NS
G = 128
K = 2
CHUNK = K * G
VB = 8192


def _table_prep(wt, vp):
    def body(wt_ref, out_ref):
        blk = jnp.concatenate(
            [wt_ref[...], jnp.zeros((DP - D, VB), jnp.float32)], axis=0)
        out_ref[...] = jnp.transpose(blk)

    return pl.pallas_call(
        body,
        grid=(vp // VB,),
        in_specs=[pl.BlockSpec((D, VB), lambda i: (0, i))],
        out_specs=pl.BlockSpec((VB, DP), lambda i: (i, 0)),
        out_shape=jax.ShapeDtypeStruct((vp, DP), jnp.float32),
    )(wt)


def _out_prep(x, hist, batch):
    rows = D * batch // 128
    lb = 4

    def body(x_ref, o_ref):
        for i in range(lb):
            c = jnp.transpose(x_ref[i])
            o_ref[pl.ds(i * rows, rows), :] = c.reshape(
                DP, batch // 128, 128)[:D].reshape(rows, 128)

    return pl.pallas_call(
        body,
        grid=(hist // lb,),
        in_specs=[pl.BlockSpec((lb, batch, DP), lambda l: (l, 0, 0))],
        out_specs=pl.BlockSpec((lb * rows, 128), lambda l: (l, 0)),
        out_shape=jax.ShapeDtypeStruct((hist * rows, 128), jnp.float32),
    )(x)


def _make_gather(n_rows):
    rows_per_w = n_rows // NW
    ng = rows_per_w // G
    nchunk = ng // K
    assert n_rows % (NW * G) == 0 and ng % (2 * K) == 0

    mesh = plsc.VectorSubcoreMesh(core_axis_name="c", subcore_axis_name="s")

    @functools.partial(
        pl.kernel,
        out_type=jax.ShapeDtypeStruct((n_rows, DP), jnp.float32),
        mesh=mesh,
        scratch_types=[
            pltpu.VMEM((ng, G), jnp.int32),
            pltpu.VMEM((CHUNK, DP), jnp.float32),
            pltpu.VMEM((CHUNK, DP), jnp.float32),
            pltpu.SemaphoreType.DMA,
            pltpu.SemaphoreType.DMA,
            pltpu.SemaphoreType.DMA,
            pltpu.SemaphoreType.DMA,
        ],
    )
    def gather_kernel(idx_hbm, table_hbm, out_hbm,
                      idx_v, buf0, buf1, gsem0, gsem1, wsem0, wsem1):
        wid = lax.axis_index("s") * NC + lax.axis_index("c")
        row0 = wid * rows_per_w

        pltpu.sync_copy(idx_hbm.at[wid], idx_v)

        def out_slice(c):
            return out_hbm.at[pl.ds(row0 + c * CHUNK, CHUNK)]

        def fire_gathers(c, buf, sem):
            return [
                pltpu.async_copy(
                    table_hbm.at[idx_v.at[c * K + j]],
                    buf.at[pl.ds(j * G, G)],
                    sem,
                )
                for j in range(K)
            ]

        def body(t, _):
            a = 2 * t

            @pl.when(t > 0)
            def _drain_prev():
                pltpu.make_async_copy(buf0, out_slice(a - 2), wsem0).wait()
                pltpu.make_async_copy(buf1, out_slice(a - 1), wsem1).wait()

            ha = fire_gathers(a, buf0, gsem0)
            hb = fire_gathers(a + 1, buf1, gsem1)
            for h in ha:
                h.wait()
            pltpu.async_copy(buf0, out_slice(a), wsem0)
            for h in hb:
                h.wait()
            pltpu.async_copy(buf1, out_slice(a + 1), wsem1)
            return 0

        lax.fori_loop(0, nchunk // 2, body, 0)
        pltpu.make_async_copy(buf0, out_slice(nchunk - 2), wsem0).wait()
        pltpu.make_async_copy(buf1, out_slice(nchunk - 1), wsem1).wait()

    return gather_kernel


def kernel(sen, word_embeddings):
    batch, hist = sen.shape
    vocab = word_embeddings.shape[0]
    vp = -(-vocab // VB) * VB
    n_rows = batch * hist
    rows_per_w = n_rows // NW

    idx = jnp.transpose(sen).reshape(NW, rows_per_w // G, G)
    table = _table_prep(jnp.transpose(word_embeddings), vp)
    out = _make_gather(n_rows)(idx, table)
    flat = _out_prep(out.reshape(hist, batch, DP), hist, batch)
    y = flat.reshape(hist, D, batch // 128, 128)
    y = jnp.transpose(y, (2, 3, 0, 1))
    return y.reshape(batch, hist, D, 1)


# speedup vs baseline: 1.8487x; 1.0090x over previous
import functools

import jax
import jax.numpy as jnp
from jax import lax
from jax.experimental import pallas as pl
from jax.experimental.pallas import tpu as pltpu
from jax.experimental.pallas import tpu_sc as plsc

D = 100
DP = 128
NC = 2
NS = 16
NW = NC * NS
G = 128
K = 2
CHUNK = K * G
VB = 16384


def _table_prep(wt, vp):
    def body(wt_ref, out_ref):
        blk = jnp.concatenate(
            [wt_ref[...], jnp.zeros((DP - D, VB), jnp.float32)], axis=0)
        out_ref[...] = jnp.transpose(blk)

    return pl.pallas_call(
        body,
        grid=(vp // VB,),
        in_specs=[pl.BlockSpec((D, VB), lambda i: (0, i))],
        out_specs=pl.BlockSpec((VB, DP), lambda i: (i, 0)),
        out_shape=jax.ShapeDtypeStruct((vp, DP), jnp.float32),
    )(wt)


def _out_prep(x, hist, batch):
    rows = D * batch // 128
    lb = 5

    def body(x_ref, o_ref):
        for i in range(lb):
            c = jnp.transpose(x_ref[i])
            o_ref[pl.ds(i * rows, rows), :] = c.reshape(
                DP, batch // 128, 128)[:D].reshape(rows, 128)

    return pl.pallas_call(
        body,
        grid=(hist // lb,),
        in_specs=[pl.BlockSpec((lb, batch, DP), lambda l: (l, 0, 0))],
        out_specs=pl.BlockSpec((lb * rows, 128), lambda l: (l, 0)),
        out_shape=jax.ShapeDtypeStruct((hist * rows, 128), jnp.float32),
    )(x)


def _make_gather(n_rows):
    rows_per_w = n_rows // NW
    ng = rows_per_w // G
    nchunk = ng // K
    assert n_rows % (NW * G) == 0 and ng % (2 * K) == 0

    mesh = plsc.VectorSubcoreMesh(core_axis_name="c", subcore_axis_name="s")

    @functools.partial(
        pl.kernel,
        out_type=jax.ShapeDtypeStruct((n_rows, DP), jnp.float32),
        mesh=mesh,
        scratch_types=[
            pltpu.VMEM((ng, G), jnp.int32),
            pltpu.VMEM((CHUNK, DP), jnp.float32),
            pltpu.VMEM((CHUNK, DP), jnp.float32),
            pltpu.SemaphoreType.DMA,
            pltpu.SemaphoreType.DMA,
            pltpu.SemaphoreType.DMA,
            pltpu.SemaphoreType.DMA,
        ],
    )
    def gather_kernel(idx_hbm, table_hbm, out_hbm,
                      idx_v, buf0, buf1, gsem0, gsem1, wsem0, wsem1):
        wid = lax.axis_index("s") * NC + lax.axis_index("c")
        row0 = wid * rows_per_w

        pltpu.sync_copy(idx_hbm.at[wid], idx_v)

        def out_slice(c):
            return out_hbm.at[pl.ds(row0 + c * CHUNK, CHUNK)]

        def fire_gathers(c, buf, sem):
            return [
                pltpu.async_copy(
                    table_hbm.at[idx_v.at[c * K + j]],
                    buf.at[pl.ds(j * G, G)],
                    sem,
                )
                for j in range(K)
            ]

        def body(t, _):
            a = 2 * t

            @pl.when(t > 0)
            def _drain_prev():
                pltpu.make_async_copy(buf0, out_slice(a - 2), wsem0).wait()
                pltpu.make_async_copy(buf1, out_slice(a - 1), wsem1).wait()

            ha = fire_gathers(a, buf0, gsem0)
            hb = fire_gathers(a + 1, buf1, gsem1)
            for h in ha:
                h.wait()
            pltpu.async_copy(buf0, out_slice(a), wsem0)
            for h in hb:
                h.wait()
            pltpu.async_copy(buf1, out_slice(a + 1), wsem1)
            return 0

        lax.fori_loop(0, nchunk // 2, body, 0)
        pltpu.make_async_copy(buf0, out_slice(nchunk - 2), wsem0).wait()
        pltpu.make_async_copy(buf1, out_slice(nchunk - 1), wsem1).wait()

    return gather_kernel


def kernel(sen, word_embeddings):
    batch, hist = sen.shape
    vocab = word_embeddings.shape[0]
    vp = -(-vocab // VB) * VB
    n_rows = batch * hist
    rows_per_w = n_rows // NW

    idx = jnp.transpose(sen).reshape(NW, rows_per_w // G, G)
    table = _table_prep(jnp.transpose(word_embeddings), vp)
    out = _make_gather(n_rows)(idx, table)
    flat = _out_prep(out.reshape(hist, batch, DP), hist, batch)
    y = flat.reshape(hist, D, batch // 128, 128)
    y = jnp.transpose(y, (2, 3, 0, 1))
    return y.reshape(batch, hist, D, 1)


# speedup vs baseline: 1.8536x; 1.0026x over previous
import functools

import jax
import jax.numpy as jnp
from jax import lax
from jax.experimental import pallas as pl
from jax.experimental.pallas import tpu as pltpu
from jax.experimental.pallas import tpu_sc as plsc

D = 100
DP = 128
NC = 2
NS = 16
NW = NC * NS
G = 128
K = 2
CHUNK = K * G
VB = 16384


def _table_prep(wt, vp):
    def body(wt_ref, out_ref):
        blk = jnp.concatenate(
            [wt_ref[...], jnp.zeros((DP - D, VB), jnp.float32)], axis=0)
        out_ref[...] = jnp.transpose(blk)

    return pl.pallas_call(
        body,
        grid=(vp // VB,),
        in_specs=[pl.BlockSpec((D, VB), lambda i: (0, i))],
        out_specs=pl.BlockSpec((VB, DP), lambda i: (i, 0)),
        out_shape=jax.ShapeDtypeStruct((vp, DP), jnp.float32),
    )(wt)


def _out_prep_piece(x, big, piece, hp, hist, batch):
    rows = D * batch // 128
    lb = 5
    nb = hp // lb

    def body(*refs):
        x_ref, o_ref = refs[-2], refs[-1]
        for i in range(lb):
            c = jnp.transpose(x_ref[i])
            o_ref[pl.ds(i * rows, rows), :] = c.reshape(
                DP, batch // 128, 128)[:D].reshape(rows, 128)

    out_spec = pl.BlockSpec((lb * rows, 128), lambda l: (piece * nb + l, 0))
    x_spec = pl.BlockSpec((lb, batch, DP), lambda l: (l, 0, 0))
    out_shape = jax.ShapeDtypeStruct((hist * rows, 128), jnp.float32)
    if big is None:
        return pl.pallas_call(
            body, grid=(nb,), in_specs=[x_spec], out_specs=out_spec,
            out_shape=out_shape)(x)
    return pl.pallas_call(
        body, grid=(nb,),
        in_specs=[pl.BlockSpec(memory_space=pl.ANY), x_spec],
        out_specs=out_spec, out_shape=out_shape,
        input_output_aliases={0: 0})(big, x)


def _make_gather(n_rows):
    rows_per_w = n_rows // NW
    ng = rows_per_w // G
    nchunk = ng // K
    assert n_rows % (NW * G) == 0 and ng % (2 * K) == 0

    mesh = plsc.VectorSubcoreMesh(core_axis_name="c", subcore_axis_name="s")

    @functools.partial(
        pl.kernel,
        out_type=jax.ShapeDtypeStruct((n_rows, DP), jnp.float32),
        mesh=mesh,
        scratch_types=[
            pltpu.VMEM((ng, G), jnp.int32),
            pltpu.VMEM((CHUNK, DP), jnp.float32),
            pltpu.VMEM((CHUNK, DP), jnp.float32),
            pltpu.SemaphoreType.DMA,
            pltpu.SemaphoreType.DMA,
            pltpu.SemaphoreType.DMA,
            pltpu.SemaphoreType.DMA,
        ],
    )
    def gather_kernel(idx_hbm, table_hbm, out_hbm,
                      idx_v, buf0, buf1, gsem0, gsem1, wsem0, wsem1):
        wid = lax.axis_index("s") * NC + lax.axis_index("c")
        row0 = wid * rows_per_w

        pltpu.sync_copy(idx_hbm.at[wid], idx_v)

        def out_slice(c):
            return out_hbm.at[pl.ds(row0 + c * CHUNK, CHUNK)]

        def fire_gathers(c, buf, sem):
            return [
                pltpu.async_copy(
                    table_hbm.at[idx_v.at[c * K + j]],
                    buf.at[pl.ds(j * G, G)],
                    sem,
                )
                for j in range(K)
            ]

        def body(t, _):
            a = 2 * t

            @pl.when(t > 0)
            def _drain_prev():
                pltpu.make_async_copy(buf0, out_slice(a - 2), wsem0).wait()
                pltpu.make_async_copy(buf1, out_slice(a - 1), wsem1).wait()

            ha = fire_gathers(a, buf0, gsem0)
            hb = fire_gathers(a + 1, buf1, gsem1)
            for h in ha:
                h.wait()
            pltpu.async_copy(buf0, out_slice(a), wsem0)
            for h in hb:
                h.wait()
            pltpu.async_copy(buf1, out_slice(a + 1), wsem1)
            return 0

        lax.fori_loop(0, nchunk // 2, body, 0)
        pltpu.make_async_copy(buf0, out_slice(nchunk - 2), wsem0).wait()
        pltpu.make_async_copy(buf1, out_slice(nchunk - 1), wsem1).wait()

    return gather_kernel


def kernel(sen, word_embeddings):
    batch, hist = sen.shape
    vocab = word_embeddings.shape[0]
    vp = -(-vocab // VB) * VB
    n_rows = batch * hist
    rows_per_w = n_rows // NW

    npiece = 2
    hp = hist // npiece
    rows_pp = n_rows // npiece
    idx = jnp.transpose(sen).reshape(npiece, NW, rows_pp // (NW * G), G)
    table = _table_prep(jnp.transpose(word_embeddings), vp)
    gather = _make_gather(rows_pp)
    flat = None
    for p in range(npiece):
        out_p = gather(idx[p], table)
        flat = _out_prep_piece(
            out_p.reshape(hp, batch, DP), flat, p, hp, hist, batch)
    y = flat.reshape(hist, D, batch // 128, 128)
    y = jnp.transpose(y, (2, 3, 0, 1))
    return y.reshape(batch, hist, D, 1)
